# Initial kernel scaffold; baseline (speedup 1.0000x reference)
import functools

import jax
import jax.numpy as jnp
from jax import lax
from jax.experimental import pallas as pl
from jax.experimental.pallas import tpu as pltpu
from jax.experimental.pallas import tpu_sc as plsc

N = 10000
D = 256
E = 160000
EPS = 1e-5

NCORES = 2
NSUB = 16
HALF = D // 2

EDGES_PER_SUB = E // NSUB
CHUNK = 200
NCH = EDGES_PER_SUB // CHUNK

ROWS_PER_SUB = N // NSUB


def _make_agg_kernel(with_cnt: bool):
    mesh = plsc.VectorSubcoreMesh(core_axis_name="c", subcore_axis_name="s")

    out_type = [jax.ShapeDtypeStruct((NCORES, N, HALF), jnp.float32)]
    if with_cnt:
        out_type.append(jax.ShapeDtypeStruct((N, 16), jnp.float32))

    scratch_types = [
        pltpu.VMEM((NCH, CHUNK), jnp.int32),
        pltpu.VMEM((NCH, CHUNK), jnp.int32),
        pltpu.VMEM((CHUNK, HALF), jnp.float32),
        pltpu.VMEM((CHUNK, 16), jnp.float32),
        pltpu.VMEM((ROWS_PER_SUB, 16), jnp.float32),
        pltpu.VMEM_SHARED((N, HALF), jnp.float32),
        pltpu.VMEM_SHARED((N, 16), jnp.float32),
        pltpu.SemaphoreType.DMA,
    ]

    def body(x2_hbm, idx_hbm, dst_hbm, *rest):
        if with_cnt:
            agg_hbm, cnt_hbm = rest[0], rest[1]
            scr = rest[2:]
        else:
            agg_hbm = rest[0]
            cnt_hbm = None
            scr = rest[1:]
        idx_v, dst_v, rows_v, ones_v, zcnt_v, agg_sp, cnt_sp, sem = scr

        cid = lax.axis_index("c")
        sid = lax.axis_index("s")
        base = sid * ROWS_PER_SUB

        pltpu.sync_copy(idx_hbm.at[cid, sid], idx_v)
        pltpu.sync_copy(dst_hbm.at[sid], dst_v)

        zero16 = jnp.zeros((16,), jnp.float32)
        one16 = jnp.ones((16,), jnp.float32)

        @pl.loop(0, CHUNK)
        def _(r):
            for j in range(HALF // 16):
                rows_v[r, pl.ds(j * 16, 16)] = zero16
            ones_v[r, pl.ds(0, 16)] = one16

        if with_cnt:
            @pl.loop(0, ROWS_PER_SUB)
            def _(r):
                zcnt_v[r, pl.ds(0, 16)] = zero16

        for off in range(0, ROWS_PER_SUB - CHUNK + 1, CHUNK):
            pltpu.sync_copy(rows_v, agg_sp.at[pl.ds(base + off, CHUNK)])
        rem = ROWS_PER_SUB % CHUNK
        if rem:
            pltpu.sync_copy(rows_v.at[pl.ds(0, rem)],
                            agg_sp.at[pl.ds(base + ROWS_PER_SUB - rem, rem)])
        if with_cnt:
            @pl.when(cid == 0)
            def _():
                pltpu.sync_copy(zcnt_v, cnt_sp.at[pl.ds(base, ROWS_PER_SUB)])

        plsc.subcore_barrier()

        @pl.loop(0, NCH)
        def _(k):
            pltpu.async_copy(x2_hbm.at[idx_v.at[k]], rows_v, sem).wait()
            pltpu.sync_copy(rows_v, agg_sp.at[dst_v.at[k]], add=True)
            if with_cnt:
                @pl.when(cid == 0)
                def _():
                    pltpu.sync_copy(ones_v, cnt_sp.at[dst_v.at[k]], add=True)

        plsc.subcore_barrier()

        pltpu.sync_copy(agg_sp.at[pl.ds(base, ROWS_PER_SUB)],
                        agg_hbm.at[cid, pl.ds(base, ROWS_PER_SUB)])
        if with_cnt:
            @pl.when(cid == 0)
            def _():
                pltpu.sync_copy(cnt_sp.at[pl.ds(base, ROWS_PER_SUB)],
                                cnt_hbm.at[pl.ds(base, ROWS_PER_SUB)])

    return pl.kernel(body, out_type=out_type, mesh=mesh,
                     scratch_types=scratch_types)


_agg_cnt = _make_agg_kernel(with_cnt=True)
_agg = _make_agg_kernel(with_cnt=False)


BN = 1000


def _tc_body(last, h_ref, a_ref, c_ref, wl_ref, bl_ref, wr_ref, g_ref, b_ref,
             o_ref):
    r = 1.0 / jnp.maximum(c_ref[...], 1.0)
    m0 = a_ref[0] * r
    m1 = a_ref[1] * r
    acc = jnp.dot(m0, wl_ref[:HALF, :], preferred_element_type=jnp.float32)
    acc += jnp.dot(m1, wl_ref[HALF:, :], preferred_element_type=jnp.float32)
    acc += jnp.dot(h_ref[...], wr_ref[...], preferred_element_type=jnp.float32)
    acc += bl_ref[...]
    if not last:
        mu = jnp.mean(acc, axis=-1, keepdims=True)
        xc = acc - mu
        var = jnp.mean(xc * xc, axis=-1, keepdims=True)
        acc = xc * lax.rsqrt(var + EPS) * g_ref[...] + b_ref[...]
        acc = jnp.maximum(acc, 0.0)
    o_ref[...] = acc


def _tc_layer(h, agg, cnt, Wl, bl, Wr, g, b, last):
    grid = N // BN
    body = functools.partial(_tc_body, last)
    return pl.pallas_call(
        body,
        grid=(grid,),
        in_specs=[
            pl.BlockSpec((BN, D), lambda i: (i, 0)),
            pl.BlockSpec((NCORES, BN, HALF), lambda i: (0, i, 0)),
            pl.BlockSpec((BN, 1), lambda i: (i, 0)),
            pl.BlockSpec((D, D), lambda i: (0, 0)),
            pl.BlockSpec((1, D), lambda i: (0, 0)),
            pl.BlockSpec((D, D), lambda i: (0, 0)),
            pl.BlockSpec((1, D), lambda i: (0, 0)),
            pl.BlockSpec((1, D), lambda i: (0, 0)),
        ],
        out_specs=pl.BlockSpec((BN, D), lambda i: (i, 0)),
        out_shape=jax.ShapeDtypeStruct((N, D), jnp.float32),
    )(h, agg, cnt, Wl, bl, Wr, g, b)


def kernel(x, edge_index, Wl0, bl0, Wr0, Wl1, bl1, Wr1, Wl2, bl2, Wr2,
           g0, b0, g1, b1):
    src = edge_index[0].astype(jnp.int32)
    dst = edge_index[1].astype(jnp.int32)

    s2 = src * 2
    idx = jnp.stack([s2, s2 + 1]).reshape(NCORES, NSUB, NCH, CHUNK)
    dstr = dst.reshape(NSUB, NCH, CHUNK)

    one_col = jnp.ones((1, D), jnp.float32)
    bl0r, bl1r, bl2r = bl0[None, :], bl1[None, :], bl2[None, :]
    g0r, b0r = g0[None, :], b0[None, :]
    g1r, b1r = g1[None, :], b1[None, :]

    agg1, cntb = _agg_cnt(x.reshape(NCORES * N, HALF), idx, dstr)
    cnt = cntb[:, :1]

    h1 = _tc_layer(x, agg1, cnt, Wl0, bl0r, Wr0, g0r, b0r, last=False)
    agg2 = _agg(h1.reshape(NCORES * N, HALF), idx, dstr)
    h2 = _tc_layer(h1, agg2, cnt, Wl1, bl1r, Wr1, g1r, b1r, last=False)
    agg3 = _agg(h2.reshape(NCORES * N, HALF), idx, dstr)
    out = _tc_layer(h2, agg3, cnt, Wl2, bl2r, Wr2, one_col, one_col, last=True)
    return out


# speedup vs baseline: 6.4870x; 6.4870x over previous
import functools

import jax
import jax.numpy as jnp
from jax import lax
from jax.experimental import pallas as pl
from jax.experimental.pallas import tpu as pltpu
from jax.experimental.pallas import tpu_sc as plsc

N = 10000
D = 256
E = 160000
EPS = 1e-5

NCORES = 2
NSUB = 16
HALF = D // 2

EDGES_PER_SUB = E // NSUB
CHUNK = 200
NCH = EDGES_PER_SUB // CHUNK

NPAD = 10240
ROWS_PER_SUB = NPAD // NSUB

_SC_PARAMS = pltpu.CompilerParams(use_tc_tiling_on_sc=False)


def _make_agg_kernel():
    mesh = plsc.VectorSubcoreMesh(core_axis_name="c", subcore_axis_name="s")

    scratch_types = [
        pltpu.VMEM((NCH, CHUNK), jnp.int32),
        pltpu.VMEM((NCH, CHUNK), jnp.int32),
        pltpu.VMEM((CHUNK, HALF), jnp.float32),
        pltpu.VMEM_SHARED((NPAD, HALF), jnp.float32),
        pltpu.SemaphoreType.DMA,
    ]

    def body(x2_hbm, idx_hbm, dst_hbm, agg_hbm, idx_v, dst_v, rows_v,
             agg_sp, sem):
        cid = lax.axis_index("c")
        sid = lax.axis_index("s")
        base = sid * ROWS_PER_SUB

        pltpu.sync_copy(idx_hbm.at[cid, sid], idx_v)
        pltpu.sync_copy(dst_hbm.at[sid], dst_v)

        zero16 = jnp.zeros((16,), jnp.float32)

        @pl.loop(0, CHUNK)
        def _(r):
            for j in range(HALF // 16):
                rows_v[r, pl.ds(j * 16, 16)] = zero16

        for off in range(0, ROWS_PER_SUB - CHUNK + 1, CHUNK):
            pltpu.sync_copy(rows_v, agg_sp.at[pl.ds(base + off, CHUNK)])
        rem = ROWS_PER_SUB % CHUNK
        if rem:
            pltpu.sync_copy(rows_v.at[pl.ds(0, rem)],
                            agg_sp.at[pl.ds(base + ROWS_PER_SUB - rem, rem)])

        plsc.subcore_barrier()

        @pl.loop(0, NCH)
        def _(k):
            pltpu.async_copy(x2_hbm.at[idx_v.at[k]], rows_v, sem).wait()
            pltpu.sync_copy(rows_v, agg_sp.at[dst_v.at[k]], add=True)

        plsc.subcore_barrier()

        pltpu.sync_copy(agg_sp.at[pl.ds(base, ROWS_PER_SUB)],
                        agg_hbm.at[cid, pl.ds(base, ROWS_PER_SUB)])

    return pl.kernel(
        body,
        out_type=jax.ShapeDtypeStruct((NCORES, NPAD, HALF), jnp.float32),
        mesh=mesh, scratch_types=scratch_types, compiler_params=_SC_PARAMS)


def _make_cnt_kernel():
    mesh = plsc.VectorSubcoreMesh(core_axis_name="c", subcore_axis_name="s")
    ncore_ch = NCH // NCORES

    scratch_types = [
        pltpu.VMEM((NCH, CHUNK), jnp.int32),
        pltpu.VMEM((CHUNK, 16), jnp.float32),
        pltpu.VMEM((ROWS_PER_SUB, 16), jnp.float32),
        pltpu.VMEM_SHARED((NPAD, 16), jnp.float32),
        pltpu.SemaphoreType.DMA,
    ]

    def body(dst_hbm, cnt_hbm, dst_v, ones_v, zcnt_v, cnt_sp, sem):
        cid = lax.axis_index("c")
        sid = lax.axis_index("s")
        base = sid * ROWS_PER_SUB

        pltpu.sync_copy(dst_hbm.at[sid], dst_v)

        zero16 = jnp.zeros((16,), jnp.float32)
        one16 = jnp.ones((16,), jnp.float32)

        @pl.loop(0, CHUNK)
        def _(r):
            ones_v[r, pl.ds(0, 16)] = one16

        @pl.loop(0, ROWS_PER_SUB)
        def _(r):
            zcnt_v[r, pl.ds(0, 16)] = zero16

        pltpu.sync_copy(zcnt_v, cnt_sp.at[pl.ds(base, ROWS_PER_SUB)])
        plsc.subcore_barrier()

        @pl.loop(0, ncore_ch)
        def _(k):
            pltpu.sync_copy(ones_v, cnt_sp.at[dst_v.at[cid * ncore_ch + k]],
                            add=True)

        plsc.subcore_barrier()
        pltpu.sync_copy(cnt_sp.at[pl.ds(base, ROWS_PER_SUB)],
                        cnt_hbm.at[cid, pl.ds(base, ROWS_PER_SUB)])

    return pl.kernel(
        body,
        out_type=jax.ShapeDtypeStruct((NCORES, NPAD, 16), jnp.float32),
        mesh=mesh, scratch_types=scratch_types, compiler_params=_SC_PARAMS)


_agg = _make_agg_kernel()
_cnt = _make_cnt_kernel()


BN = 1000


def _tc_body(last, h_ref, a_ref, c_ref, wl_ref, bl_ref, wr_ref, g_ref, b_ref,
             o_ref):
    cnt = c_ref[0] + c_ref[1]
    r = 1.0 / jnp.maximum(cnt, 1.0)
    m0 = a_ref[0] * r
    m1 = a_ref[1] * r
    acc = jnp.dot(m0, wl_ref[:HALF, :], preferred_element_type=jnp.float32)
    acc += jnp.dot(m1, wl_ref[HALF:, :], preferred_element_type=jnp.float32)
    acc += jnp.dot(h_ref[...], wr_ref[...], preferred_element_type=jnp.float32)
    acc += bl_ref[...]
    if not last:
        mu = jnp.mean(acc, axis=-1, keepdims=True)
        xc = acc - mu
        var = jnp.mean(xc * xc, axis=-1, keepdims=True)
        acc = xc * lax.rsqrt(var + EPS) * g_ref[...] + b_ref[...]
        acc = jnp.maximum(acc, 0.0)
    o_ref[...] = acc


def _tc_layer(h, agg, cnt, Wl, bl, Wr, g, b, last):
    grid = N // BN
    body = functools.partial(_tc_body, last)
    return pl.pallas_call(
        body,
        grid=(grid,),
        in_specs=[
            pl.BlockSpec((BN, D), lambda i: (i, 0)),
            pl.BlockSpec((NCORES, BN, HALF), lambda i: (0, i, 0)),
            pl.BlockSpec((NCORES, BN, 1), lambda i: (0, i, 0)),
            pl.BlockSpec((D, D), lambda i: (0, 0)),
            pl.BlockSpec((1, D), lambda i: (0, 0)),
            pl.BlockSpec((D, D), lambda i: (0, 0)),
            pl.BlockSpec((1, D), lambda i: (0, 0)),
            pl.BlockSpec((1, D), lambda i: (0, 0)),
        ],
        out_specs=pl.BlockSpec((BN, D), lambda i: (i, 0)),
        out_shape=jax.ShapeDtypeStruct((N, D), jnp.float32),
    )(h, agg, cnt, Wl, bl, Wr, g, b)


def kernel(x, edge_index, Wl0, bl0, Wr0, Wl1, bl1, Wr1, Wl2, bl2, Wr2,
           g0, b0, g1, b1):
    src = edge_index[0].astype(jnp.int32)
    dst = edge_index[1].astype(jnp.int32)

    s2 = src * 2
    idx = jnp.stack([s2, s2 + 1]).reshape(NCORES, NSUB, NCH, CHUNK)
    dstr = dst.reshape(NSUB, NCH, CHUNK)

    one_col = jnp.ones((1, D), jnp.float32)
    bl0r, bl1r, bl2r = bl0[None, :], bl1[None, :], bl2[None, :]
    g0r, b0r = g0[None, :], b0[None, :]
    g1r, b1r = g1[None, :], b1[None, :]

    cntb = _cnt(dstr)
    cnt = cntb[:, :, :1]

    agg1 = _agg(x.reshape(NCORES * N, HALF), idx, dstr)
    h1 = _tc_layer(x, agg1, cnt, Wl0, bl0r, Wr0, g0r, b0r, last=False)
    agg2 = _agg(h1.reshape(NCORES * N, HALF), idx, dstr)
    h2 = _tc_layer(h1, agg2, cnt, Wl1, bl1r, Wr1, g1r, b1r, last=False)
    agg3 = _agg(h2.reshape(NCORES * N, HALF), idx, dstr)
    out = _tc_layer(h2, agg3, cnt, Wl2, bl2r, Wr2, one_col, one_col, last=True)
    return out


# speedup vs baseline: 8.0241x; 1.2369x over previous
import functools

import jax
import jax.numpy as jnp
from jax import lax
from jax.experimental import pallas as pl
from jax.experimental.pallas import tpu as pltpu
from jax.experimental.pallas import tpu_sc as plsc

N = 10000
D = 256
E = 160000
EPS = 1e-5

NCORES = 2
NSUB = 16
HALF = D // 2

EDGES_PER_SUB = E // NSUB
CHUNK = 80
NCH = EDGES_PER_SUB // CHUNK

NPAD = 10240
ROWS_PER_SUB = NPAD // NSUB

_SC_PARAMS = pltpu.CompilerParams(use_tc_tiling_on_sc=False)


def _make_agg_kernel():
    mesh = plsc.VectorSubcoreMesh(core_axis_name="c", subcore_axis_name="s")

    scratch_types = [
        pltpu.VMEM((NCH, CHUNK), jnp.int32),
        pltpu.VMEM((NCH, CHUNK), jnp.int32),
        pltpu.VMEM((CHUNK, HALF), jnp.float32),
        pltpu.VMEM((CHUNK, HALF), jnp.float32),
        pltpu.VMEM_SHARED((NPAD, HALF), jnp.float32),
        pltpu.SemaphoreType.DMA,
        pltpu.SemaphoreType.DMA,
    ]

    def body(x2_hbm, idx_hbm, dst_hbm, agg_hbm, idx_v, dst_v, rows_a,
             rows_b, agg_sp, sem_a, sem_b):
        cid = lax.axis_index("c")
        sid = lax.axis_index("s")
        base = sid * ROWS_PER_SUB

        pltpu.sync_copy(idx_hbm.at[cid, sid], idx_v)
        pltpu.sync_copy(dst_hbm.at[sid], dst_v)

        pltpu.async_copy(x2_hbm.at[idx_v.at[0]], rows_a, sem_a)

        zero16 = jnp.zeros((16,), jnp.float32)

        @pl.loop(0, CHUNK)
        def _(r):
            for j in range(HALF // 16):
                rows_b[r, pl.ds(j * 16, 16)] = zero16

        for off in range(0, ROWS_PER_SUB - CHUNK + 1, CHUNK):
            pltpu.sync_copy(rows_b, agg_sp.at[pl.ds(base + off, CHUNK)])
        rem = ROWS_PER_SUB % CHUNK
        if rem:
            pltpu.sync_copy(rows_b.at[pl.ds(0, rem)],
                            agg_sp.at[pl.ds(base + ROWS_PER_SUB - rem, rem)])

        plsc.subcore_barrier()

        @pl.loop(0, NCH - 1, step=2)
        def _(k):
            pltpu.async_copy(x2_hbm.at[idx_v.at[k + 1]], rows_b, sem_b)
            pltpu.make_async_copy(x2_hbm.at[idx_v.at[k]], rows_a, sem_a).wait()
            pltpu.sync_copy(rows_a, agg_sp.at[dst_v.at[k]], add=True)

            pltpu.async_copy(x2_hbm.at[idx_v.at[k + 2]], rows_a, sem_a)
            pltpu.make_async_copy(x2_hbm.at[idx_v.at[k + 1]], rows_b,
                                  sem_b).wait()
            pltpu.sync_copy(rows_b, agg_sp.at[dst_v.at[k + 1]], add=True)

        pltpu.make_async_copy(x2_hbm.at[idx_v.at[NCH - 1]], rows_a,
                              sem_a).wait()
        pltpu.sync_copy(rows_a, agg_sp.at[dst_v.at[NCH - 1]], add=True)

        plsc.subcore_barrier()

        pltpu.sync_copy(agg_sp.at[pl.ds(base, ROWS_PER_SUB)],
                        agg_hbm.at[cid, pl.ds(base, ROWS_PER_SUB)])

    return pl.kernel(
        body,
        out_type=jax.ShapeDtypeStruct((NCORES, NPAD, HALF), jnp.float32),
        mesh=mesh, scratch_types=scratch_types, compiler_params=_SC_PARAMS)


def _make_cnt_kernel():
    mesh = plsc.VectorSubcoreMesh(core_axis_name="c", subcore_axis_name="s")
    split = NCH // NCORES + 1

    scratch_types = [
        pltpu.VMEM((NCH, CHUNK), jnp.int32),
        pltpu.VMEM((CHUNK, 16), jnp.float32),
        pltpu.VMEM((ROWS_PER_SUB, 16), jnp.float32),
        pltpu.VMEM_SHARED((NPAD, 16), jnp.float32),
        pltpu.SemaphoreType.DMA,
    ]

    def body(dst_hbm, cnt_hbm, dst_v, ones_v, zcnt_v, cnt_sp, sem):
        cid = lax.axis_index("c")
        sid = lax.axis_index("s")
        base = sid * ROWS_PER_SUB

        pltpu.sync_copy(dst_hbm.at[sid], dst_v)

        zero16 = jnp.zeros((16,), jnp.float32)
        one16 = jnp.ones((16,), jnp.float32)

        @pl.loop(0, CHUNK)
        def _(r):
            ones_v[r, pl.ds(0, 16)] = one16

        @pl.loop(0, ROWS_PER_SUB)
        def _(r):
            zcnt_v[r, pl.ds(0, 16)] = zero16

        pltpu.sync_copy(zcnt_v, cnt_sp.at[pl.ds(base, ROWS_PER_SUB)])
        plsc.subcore_barrier()

        lo = cid * split
        hi = lo + jnp.where(cid == 0, split, NCH - split)

        @pl.loop(lo, hi)
        def _(k):
            pltpu.sync_copy(ones_v, cnt_sp.at[dst_v.at[k]], add=True)

        plsc.subcore_barrier()
        pltpu.sync_copy(cnt_sp.at[pl.ds(base, ROWS_PER_SUB)],
                        cnt_hbm.at[cid, pl.ds(base, ROWS_PER_SUB)])

    return pl.kernel(
        body,
        out_type=jax.ShapeDtypeStruct((NCORES, NPAD, 16), jnp.float32),
        mesh=mesh, scratch_types=scratch_types, compiler_params=_SC_PARAMS)


_agg = _make_agg_kernel()
_cnt = _make_cnt_kernel()


BN = 1000


def _tc_body(last, h_ref, a_ref, c_ref, wl_ref, bl_ref, wr_ref, g_ref, b_ref,
             o_ref):
    cnt = c_ref[0] + c_ref[1]
    r = 1.0 / jnp.maximum(cnt, 1.0)
    m0 = a_ref[0] * r
    m1 = a_ref[1] * r
    acc = jnp.dot(m0, wl_ref[:HALF, :], preferred_element_type=jnp.float32)
    acc += jnp.dot(m1, wl_ref[HALF:, :], preferred_element_type=jnp.float32)
    acc += jnp.dot(h_ref[...], wr_ref[...], preferred_element_type=jnp.float32)
    acc += bl_ref[...]
    if not last:
        mu = jnp.mean(acc, axis=-1, keepdims=True)
        xc = acc - mu
        var = jnp.mean(xc * xc, axis=-1, keepdims=True)
        acc = xc * lax.rsqrt(var + EPS) * g_ref[...] + b_ref[...]
        acc = jnp.maximum(acc, 0.0)
    o_ref[...] = acc


def _tc_layer(h, agg, cnt, Wl, bl, Wr, g, b, last):
    grid = N // BN
    body = functools.partial(_tc_body, last)
    return pl.pallas_call(
        body,
        grid=(grid,),
        in_specs=[
            pl.BlockSpec((BN, D), lambda i: (i, 0)),
            pl.BlockSpec((NCORES, BN, HALF), lambda i: (0, i, 0)),
            pl.BlockSpec((NCORES, BN, 1), lambda i: (0, i, 0)),
            pl.BlockSpec((D, D), lambda i: (0, 0)),
            pl.BlockSpec((1, D), lambda i: (0, 0)),
            pl.BlockSpec((D, D), lambda i: (0, 0)),
            pl.BlockSpec((1, D), lambda i: (0, 0)),
            pl.BlockSpec((1, D), lambda i: (0, 0)),
        ],
        out_specs=pl.BlockSpec((BN, D), lambda i: (i, 0)),
        out_shape=jax.ShapeDtypeStruct((N, D), jnp.float32),
    )(h, agg, cnt, Wl, bl, Wr, g, b)


def kernel(x, edge_index, Wl0, bl0, Wr0, Wl1, bl1, Wr1, Wl2, bl2, Wr2,
           g0, b0, g1, b1):
    src = edge_index[0].astype(jnp.int32)
    dst = edge_index[1].astype(jnp.int32)

    s2 = src * 2
    idx = jnp.stack([s2, s2 + 1]).reshape(NCORES, NSUB, NCH, CHUNK)
    dstr = dst.reshape(NSUB, NCH, CHUNK)

    one_col = jnp.ones((1, D), jnp.float32)
    bl0r, bl1r, bl2r = bl0[None, :], bl1[None, :], bl2[None, :]
    g0r, b0r = g0[None, :], b0[None, :]
    g1r, b1r = g1[None, :], b1[None, :]

    cntb = _cnt(dstr)
    cnt = cntb[:, :, :1]

    agg1 = _agg(x.reshape(NCORES * N, HALF), idx, dstr)
    h1 = _tc_layer(x, agg1, cnt, Wl0, bl0r, Wr0, g0r, b0r, last=False)
    agg2 = _agg(h1.reshape(NCORES * N, HALF), idx, dstr)
    h2 = _tc_layer(h1, agg2, cnt, Wl1, bl1r, Wr1, g1r, b1r, last=False)
    agg3 = _agg(h2.reshape(NCORES * N, HALF), idx, dstr)
    out = _tc_layer(h2, agg3, cnt, Wl2, bl2r, Wr2, one_col, one_col, last=True)
    return out


# speedup vs baseline: 8.5302x; 1.0631x over previous
import functools

import jax
import jax.numpy as jnp
from jax import lax
from jax.experimental import pallas as pl
from jax.experimental.pallas import tpu as pltpu
from jax.experimental.pallas import tpu_sc as plsc

N = 10000
D = 256
E = 160000
EPS = 1e-5

NCORES = 2
NSUB = 16
HALF = D // 2

EDGES_PER_SUB = E // NSUB
CHUNK = 80
NCH = EDGES_PER_SUB // CHUNK

NPAD = 10240
ROWS_PER_SUB = NPAD // NSUB

_SC_PARAMS = pltpu.CompilerParams(use_tc_tiling_on_sc=False)


def _make_agg_kernel():
    mesh = plsc.VectorSubcoreMesh(core_axis_name="c", subcore_axis_name="s")

    scratch_types = [
        pltpu.VMEM((NCH, CHUNK), jnp.int32),
        pltpu.VMEM((NCH, CHUNK), jnp.int32),
        pltpu.VMEM((CHUNK, HALF), jnp.float32),
        pltpu.VMEM((CHUNK, HALF), jnp.float32),
        pltpu.VMEM_SHARED((NPAD, HALF), jnp.float32),
        pltpu.SemaphoreType.DMA,
        pltpu.SemaphoreType.DMA,
    ]

    def body(x2_hbm, idx_hbm, dst_hbm, agg_hbm, idx_v, dst_v, rows_a,
             rows_b, agg_sp, sem_a, sem_b):
        cid = lax.axis_index("c")
        sid = lax.axis_index("s")
        base = sid * ROWS_PER_SUB

        pltpu.sync_copy(idx_hbm.at[sid], idx_v)
        pltpu.sync_copy(dst_hbm.at[sid], dst_v)

        xh = x2_hbm.at[cid]

        pltpu.async_copy(xh.at[idx_v.at[0]], rows_a, sem_a)

        zero16 = jnp.zeros((16,), jnp.float32)

        @pl.loop(0, CHUNK)
        def _(r):
            for j in range(HALF // 16):
                rows_b[r, pl.ds(j * 16, 16)] = zero16

        for off in range(0, ROWS_PER_SUB - CHUNK + 1, CHUNK):
            pltpu.sync_copy(rows_b, agg_sp.at[pl.ds(base + off, CHUNK)])
        rem = ROWS_PER_SUB % CHUNK
        if rem:
            pltpu.sync_copy(rows_b.at[pl.ds(0, rem)],
                            agg_sp.at[pl.ds(base + ROWS_PER_SUB - rem, rem)])

        plsc.subcore_barrier()

        @pl.loop(0, NCH - 1, step=2)
        def _(k):
            pltpu.async_copy(xh.at[idx_v.at[k + 1]], rows_b, sem_b)
            pltpu.make_async_copy(xh.at[idx_v.at[k]], rows_a, sem_a).wait()
            pltpu.sync_copy(rows_a, agg_sp.at[dst_v.at[k]], add=True)

            pltpu.async_copy(xh.at[idx_v.at[k + 2]], rows_a, sem_a)
            pltpu.make_async_copy(xh.at[idx_v.at[k + 1]], rows_b,
                                  sem_b).wait()
            pltpu.sync_copy(rows_b, agg_sp.at[dst_v.at[k + 1]], add=True)

        pltpu.make_async_copy(xh.at[idx_v.at[NCH - 1]], rows_a,
                              sem_a).wait()
        pltpu.sync_copy(rows_a, agg_sp.at[dst_v.at[NCH - 1]], add=True)

        plsc.subcore_barrier()

        pltpu.sync_copy(agg_sp.at[pl.ds(base, ROWS_PER_SUB)],
                        agg_hbm.at[cid, pl.ds(base, ROWS_PER_SUB)])

    return pl.kernel(
        body,
        out_type=jax.ShapeDtypeStruct((NCORES, NPAD, HALF), jnp.float32),
        mesh=mesh, scratch_types=scratch_types, compiler_params=_SC_PARAMS)


def _make_cnt_kernel():
    mesh = plsc.VectorSubcoreMesh(core_axis_name="c", subcore_axis_name="s")
    split = NCH // NCORES + 1

    scratch_types = [
        pltpu.VMEM((NCH, CHUNK), jnp.int32),
        pltpu.VMEM((CHUNK, 16), jnp.float32),
        pltpu.VMEM((ROWS_PER_SUB, 16), jnp.float32),
        pltpu.VMEM_SHARED((NPAD, 16), jnp.float32),
        pltpu.SemaphoreType.DMA,
    ]

    def body(dst_hbm, cnt_hbm, dst_v, ones_v, zcnt_v, cnt_sp, sem):
        cid = lax.axis_index("c")
        sid = lax.axis_index("s")
        base = sid * ROWS_PER_SUB

        pltpu.sync_copy(dst_hbm.at[sid], dst_v)

        zero16 = jnp.zeros((16,), jnp.float32)
        one16 = jnp.ones((16,), jnp.float32)

        @pl.loop(0, CHUNK)
        def _(r):
            ones_v[r, pl.ds(0, 16)] = one16

        @pl.loop(0, ROWS_PER_SUB)
        def _(r):
            zcnt_v[r, pl.ds(0, 16)] = zero16

        pltpu.sync_copy(zcnt_v, cnt_sp.at[pl.ds(base, ROWS_PER_SUB)])
        plsc.subcore_barrier()

        lo = cid * split
        hi = lo + jnp.where(cid == 0, split, NCH - split)

        @pl.loop(lo, hi)
        def _(k):
            pltpu.sync_copy(ones_v, cnt_sp.at[dst_v.at[k]], add=True)

        plsc.subcore_barrier()
        pltpu.sync_copy(cnt_sp.at[pl.ds(base, ROWS_PER_SUB)],
                        cnt_hbm.at[cid, pl.ds(base, ROWS_PER_SUB)])

    return pl.kernel(
        body,
        out_type=jax.ShapeDtypeStruct((NCORES, NPAD, 16), jnp.float32),
        mesh=mesh, scratch_types=scratch_types, compiler_params=_SC_PARAMS)


_agg = _make_agg_kernel()
_cnt = _make_cnt_kernel()


BN = 1000


def _tc_body(last, h_ref, a_ref, c_ref, wl_ref, bl_ref, wr_ref, g_ref, b_ref,
             o_ref):
    cnt = c_ref[0] + c_ref[1]
    r = 1.0 / jnp.maximum(cnt, 1.0)
    m0 = a_ref[0] * r
    m1 = a_ref[1] * r
    acc = jnp.dot(m0, wl_ref[:HALF, :], preferred_element_type=jnp.float32)
    acc += jnp.dot(m1, wl_ref[HALF:, :], preferred_element_type=jnp.float32)
    acc += jnp.dot(h_ref[0], wr_ref[:HALF, :],
                   preferred_element_type=jnp.float32)
    acc += jnp.dot(h_ref[1], wr_ref[HALF:, :],
                   preferred_element_type=jnp.float32)
    acc += bl_ref[...]
    if not last:
        mu = jnp.mean(acc, axis=-1, keepdims=True)
        xc = acc - mu
        var = jnp.mean(xc * xc, axis=-1, keepdims=True)
        acc = xc * lax.rsqrt(var + EPS) * g_ref[...] + b_ref[...]
        acc = jnp.maximum(acc, 0.0)
    if last:
        o_ref[...] = acc
    else:
        o_ref[0] = acc[:, :HALF]
        o_ref[1] = acc[:, HALF:]


def _tc_layer(h, agg, cnt, Wl, bl, Wr, g, b, last):
    grid = N // BN
    body = functools.partial(_tc_body, last)
    if last:
        out_spec = pl.BlockSpec((BN, D), lambda i: (i, 0))
        out_shape = jax.ShapeDtypeStruct((N, D), jnp.float32)
    else:
        out_spec = pl.BlockSpec((NCORES, BN, HALF), lambda i: (0, i, 0))
        out_shape = jax.ShapeDtypeStruct((NCORES, N, HALF), jnp.float32)
    return pl.pallas_call(
        body,
        grid=(grid,),
        in_specs=[
            pl.BlockSpec((NCORES, BN, HALF), lambda i: (0, i, 0)),
            pl.BlockSpec((NCORES, BN, HALF), lambda i: (0, i, 0)),
            pl.BlockSpec((NCORES, BN, 1), lambda i: (0, i, 0)),
            pl.BlockSpec((D, D), lambda i: (0, 0)),
            pl.BlockSpec((1, D), lambda i: (0, 0)),
            pl.BlockSpec((D, D), lambda i: (0, 0)),
            pl.BlockSpec((1, D), lambda i: (0, 0)),
            pl.BlockSpec((1, D), lambda i: (0, 0)),
        ],
        out_specs=out_spec,
        out_shape=out_shape,
    )(h, agg, cnt, Wl, bl, Wr, g, b)


def kernel(x, edge_index, Wl0, bl0, Wr0, Wl1, bl1, Wr1, Wl2, bl2, Wr2,
           g0, b0, g1, b1):
    src = edge_index[0].astype(jnp.int32)
    dst = edge_index[1].astype(jnp.int32)

    idx = src.reshape(NSUB, NCH, CHUNK)
    dstr = dst.reshape(NSUB, NCH, CHUNK)
    x2 = jnp.stack([x[:, :HALF], x[:, HALF:]])

    one_col = jnp.ones((1, D), jnp.float32)
    bl0r, bl1r, bl2r = bl0[None, :], bl1[None, :], bl2[None, :]
    g0r, b0r = g0[None, :], b0[None, :]
    g1r, b1r = g1[None, :], b1[None, :]

    cntb = _cnt(dstr)
    cnt = cntb[:, :, :1]

    agg1 = _agg(x2, idx, dstr)
    h1 = _tc_layer(x2, agg1, cnt, Wl0, bl0r, Wr0, g0r, b0r, last=False)
    agg2 = _agg(h1, idx, dstr)
    h2 = _tc_layer(h1, agg2, cnt, Wl1, bl1r, Wr1, g1r, b1r, last=False)
    agg3 = _agg(h2, idx, dstr)
    out = _tc_layer(h2, agg3, cnt, Wl2, bl2r, Wr2, one_col, one_col, last=True)
    return out
